# Initial kernel scaffold; baseline (speedup 1.0000x reference)
import jax
import jax.numpy as jnp
from jax.experimental import pallas as pl


def kernel(x, W, b):
    raise NotImplementedError("write your pallas kernel here")


# speedup vs baseline: 2.2307x; 2.2307x over previous
import jax
import jax.numpy as jnp
from jax.experimental import pallas as pl

_TM = 512


def _mm_filter_kernel(x_ref, w_ref, b_ref, o_ref):
    acc = jax.lax.dot_general(
        x_ref[...], w_ref[...],
        dimension_numbers=(((1,), (1,)), ((), ())),
        preferred_element_type=jnp.float32,
    )
    out = acc + b_ref[...]
    row_ok = jnp.all(jnp.isfinite(out), axis=1, keepdims=True)
    o_ref[...] = jnp.where(row_ok, out, jnp.zeros_like(out))


def kernel(x, W, b):
    M, K = x.shape
    N = W.shape[0]
    b2 = b.reshape(1, N)
    return pl.pallas_call(
        _mm_filter_kernel,
        grid=(M // _TM,),
        in_specs=[
            pl.BlockSpec((_TM, K), lambda i: (i, 0)),
            pl.BlockSpec((N, K), lambda i: (0, 0)),
            pl.BlockSpec((1, N), lambda i: (0, 0)),
        ],
        out_specs=pl.BlockSpec((_TM, N), lambda i: (i, 0)),
        out_shape=jax.ShapeDtypeStruct((M, N), jnp.float32),
    )(x, W, b2)
